# baseline (speedup 1.0000x reference)
import jax
import jax.numpy as jnp
from jax import lax
from jax.experimental import pallas as pl
from jax.experimental.pallas import tpu as pltpu
from jax.experimental.pallas import tpu_sc as plsc

BATCH = 32
POINT_NUM = 6890
MAX_NB = 8
LANES = 16
NB_SLOTS = MAX_NB - 1
NB_PAD = 6896
GROUPS = NB_PAD // LANES
W = POINT_NUM * 3
TOT = BATCH * W
L_DMA = 20688
ZSLOT = L_DMA
BUF = L_DMA + LANES
BIG = 1 << 22
INV_N = 1.0 / (BATCH * POINT_NUM * 3)
SUB_GROUPS = L_DMA // LANES


def _sc_body(gt_hbm, pr_hbm, nbt_hbm, nn_hbm, out_hbm,
             d_buf, t_buf, nbt_v, nn_v, o_v, sem_a, sem_b):
    b = lax.axis_index("s") * 2 + lax.axis_index("c")
    start = b * W
    off = jnp.minimum((start // 8) * 8, TOT - L_DMA)
    off = pl.multiple_of(off, 8)
    shift = start - off

    cp1 = pltpu.make_async_copy(gt_hbm.at[pl.ds(off, L_DMA)],
                                d_buf.at[pl.ds(0, L_DMA)], sem_a)
    cp2 = pltpu.make_async_copy(pr_hbm.at[pl.ds(off, L_DMA)], t_buf, sem_a)
    cp3 = pltpu.make_async_copy(nn_hbm, nn_v, sem_a)
    cp4 = pltpu.make_async_copy(nbt_hbm, nbt_v, sem_b)
    cp1.start()
    cp2.start()
    cp3.start()
    cp4.start()
    cp1.wait()
    cp2.wait()
    cp3.wait()

    def sub_body(g, carry):
        s = pl.ds(g * LANES, LANES)
        d_buf[s] = d_buf[s] - t_buf[s]
        return carry

    lax.fori_loop(0, SUB_GROUPS, sub_body, 0, unroll=4)
    d_buf[pl.ds(ZSLOT, LANES)] = jnp.zeros((LANES,), jnp.float32)
    cp4.wait()

    limit3 = (POINT_NUM - 1) * 3 + shift

    def main_body(g, carry):
        lap, geo, base3 = carry
        s = pl.ds(g * LANES, LANES)
        nnv = nn_v[s]
        c3 = jnp.minimum(base3, limit3)
        x = plsc.load_gather(d_buf, [c3])
        y = plsc.load_gather(d_buf, [c3 + 1])
        z = plsc.load_gather(d_buf, [c3 + 2])
        ax = x * nnv
        ay = y * nnv
        az = z * nnv
        gabs = jnp.abs(x) + jnp.abs(y) + jnp.abs(z)
        geo = geo + jnp.where(base3 <= limit3, gabs, 0.0)
        for n in range(NB_SLOTS):
            i3 = nbt_v[n, s]
            si = jnp.minimum(i3 + shift, ZSLOT)
            ax = ax - plsc.load_gather(d_buf, [si])
            ay = ay - plsc.load_gather(d_buf, [si + 1])
            az = az - plsc.load_gather(d_buf, [si + 2])
        lap = lap + jnp.abs(ax) + jnp.abs(ay) + jnp.abs(az)
        return lap, geo, base3 + 3 * LANES

    zero = jnp.zeros((LANES,), jnp.float32)
    base0 = lax.iota(jnp.int32, LANES) * 3 + shift
    lap, geo, _ = lax.fori_loop(0, GROUPS, main_body, (zero, zero, base0))
    o_v[...] = (lap + geo) * INV_N
    pltpu.sync_copy(o_v, out_hbm.at[pl.ds(b * LANES, LANES)])


@jax.jit
def _mcloss(gt_f, pr_f, nbt3, nn_p):
    call = pl.kernel(
        _sc_body,
        out_type=jax.ShapeDtypeStruct((BATCH * LANES,), jnp.float32),
        mesh=plsc.VectorSubcoreMesh(
            core_axis_name="c", subcore_axis_name="s",
            num_cores=2, num_subcores=16),
        compiler_params=pltpu.CompilerParams(needs_layout_passes=False),
        scratch_types=[
            pltpu.VMEM((BUF,), jnp.float32),
            pltpu.VMEM((L_DMA,), jnp.float32),
            pltpu.VMEM((NB_SLOTS, NB_PAD), jnp.int32),
            pltpu.VMEM((NB_PAD,), jnp.float32),
            pltpu.VMEM((LANES,), jnp.float32),
            pltpu.SemaphoreType.DMA,
            pltpu.SemaphoreType.DMA,
        ],
    )
    parts = call(gt_f, pr_f, nbt3, nn_p)
    return jnp.sum(parts)


def kernel(gt_pc, predict_pc, neighbor_id_lstlst, neighbor_num_lst):
    pad = NB_PAD - POINT_NUM
    gt_f = gt_pc.reshape(-1)
    pr_f = predict_pc.reshape(-1)
    nb = neighbor_id_lstlst[:, 1:]
    nbt3 = jnp.pad(jnp.where(nb < POINT_NUM, nb * 3, BIG).T,
                   ((0, 0), (0, pad)), constant_values=BIG)
    nn_p = jnp.pad(neighbor_num_lst, (0, pad))
    return _mcloss(gt_f, pr_f, nbt3, nn_p)


# speedup vs baseline: 5.8986x; 5.8986x over previous
import jax
import jax.numpy as jnp
from jax import lax
from jax.experimental import pallas as pl
from jax.experimental.pallas import tpu as pltpu
from jax.experimental.pallas import tpu_sc as plsc

BATCH = 32
POINT_NUM = 6890
MAX_NB = 8
LANES = 16
NB_SLOTS = MAX_NB - 1
PADDED = 6896
GROUPS = PADDED // LANES
INV_N = 1.0 / (BATCH * POINT_NUM * 3)


def _sc_body(gt_hbm, pr_hbm, nbt_hbm, nn_hbm, out_hbm,
             d0, d1, d2, t0, t1, t2, nbt_v, nn_v, o_v, sem_a, sem_b):
    b = lax.axis_index("s") * 2 + lax.axis_index("c")
    base = b * 3 * PADDED

    cps = [
        pltpu.make_async_copy(gt_hbm.at[pl.ds(base + 0 * PADDED, PADDED)], d0,
                              sem_a),
        pltpu.make_async_copy(gt_hbm.at[pl.ds(base + 1 * PADDED, PADDED)], d1,
                              sem_a),
        pltpu.make_async_copy(gt_hbm.at[pl.ds(base + 2 * PADDED, PADDED)], d2,
                              sem_a),
        pltpu.make_async_copy(pr_hbm.at[pl.ds(base + 0 * PADDED, PADDED)], t0,
                              sem_a),
        pltpu.make_async_copy(pr_hbm.at[pl.ds(base + 1 * PADDED, PADDED)], t1,
                              sem_a),
        pltpu.make_async_copy(pr_hbm.at[pl.ds(base + 2 * PADDED, PADDED)], t2,
                              sem_a),
    ]
    cpn = [
        pltpu.make_async_copy(nbt_hbm, nbt_v, sem_b),
        pltpu.make_async_copy(nn_hbm, nn_v, sem_b),
    ]
    for c in cps:
        c.start()
    for c in cpn:
        c.start()
    for c in cps:
        c.wait()

    def sub_body(g, carry):
        s = pl.ds(g * LANES, LANES)
        d0[s] = d0[s] - t0[s]
        d1[s] = d1[s] - t1[s]
        d2[s] = d2[s] - t2[s]
        return carry

    lax.fori_loop(0, GROUPS, sub_body, 0, unroll=4)
    for c in cpn:
        c.wait()

    def main_body(g, carry):
        lap, geo = carry
        s = pl.ds(g * LANES, LANES)
        nnv = nn_v[s]
        x = d0[s]
        y = d1[s]
        z = d2[s]
        geo = geo + jnp.abs(x) + jnp.abs(y) + jnp.abs(z)
        idx = [nbt_v[n, s] for n in range(NB_SLOTS)]
        gx = [plsc.load_gather(d0, [i]) for i in idx]
        gy = [plsc.load_gather(d1, [i]) for i in idx]
        gz = [plsc.load_gather(d2, [i]) for i in idx]

        def tree7(g):
            return ((g[0] + g[1]) + (g[2] + g[3])) + ((g[4] + g[5]) + g[6])

        ax = x * nnv - tree7(gx)
        ay = y * nnv - tree7(gy)
        az = z * nnv - tree7(gz)
        lap = lap + jnp.abs(ax) + jnp.abs(ay) + jnp.abs(az)
        return lap, geo

    zero = jnp.zeros((LANES,), jnp.float32)
    lap, geo = lax.fori_loop(0, GROUPS, main_body, (zero, zero), unroll=2)
    o_v[...] = (lap + geo) * INV_N
    pltpu.sync_copy(o_v, out_hbm.at[pl.ds(b * LANES, LANES)])


@jax.jit
def _mcloss(gt_t, pr_t, nbt, nn_p):
    call = pl.kernel(
        _sc_body,
        out_type=jax.ShapeDtypeStruct((BATCH * LANES,), jnp.float32),
        mesh=plsc.VectorSubcoreMesh(
            core_axis_name="c", subcore_axis_name="s",
            num_cores=2, num_subcores=16),
        compiler_params=pltpu.CompilerParams(needs_layout_passes=False),
        scratch_types=[
            pltpu.VMEM((PADDED,), jnp.float32),
            pltpu.VMEM((PADDED,), jnp.float32),
            pltpu.VMEM((PADDED,), jnp.float32),
            pltpu.VMEM((PADDED,), jnp.float32),
            pltpu.VMEM((PADDED,), jnp.float32),
            pltpu.VMEM((PADDED,), jnp.float32),
            pltpu.VMEM((NB_SLOTS, PADDED), jnp.int32),
            pltpu.VMEM((PADDED,), jnp.float32),
            pltpu.VMEM((LANES,), jnp.float32),
            pltpu.SemaphoreType.DMA,
            pltpu.SemaphoreType.DMA,
        ],
    )
    parts = call(gt_t, pr_t, nbt, nn_p)
    return jnp.sum(parts)


def kernel(gt_pc, predict_pc, neighbor_id_lstlst, neighbor_num_lst):
    pad = PADDED - POINT_NUM
    gt_t = jnp.pad(jnp.transpose(gt_pc, (0, 2, 1)),
                   ((0, 0), (0, 0), (0, pad))).reshape(-1)
    pr_t = jnp.pad(jnp.transpose(predict_pc, (0, 2, 1)),
                   ((0, 0), (0, 0), (0, pad))).reshape(-1)
    nbt = jnp.pad(jnp.transpose(neighbor_id_lstlst[:, 1:], (1, 0)),
                  ((0, 0), (0, pad)), constant_values=POINT_NUM)
    nn_p = jnp.pad(neighbor_num_lst, (0, pad))
    return _mcloss(gt_t, pr_t, nbt, nn_p)
